# baseline (speedup 1.0000x reference)
import jax
import jax.numpy as jnp
from jax import lax
from jax.experimental import pallas as pl
from jax.experimental.pallas import tpu as pltpu
from jax.experimental.pallas import tpu_sc as plsc

NTOKEN = 1000000
EMB_DIM = 64
BATCH = 16384
HIST_LEN = 50

NC = 2
NS = 16
NW = NC * NS

B = BATCH * HIST_LEN
PER_W = B // NW
CHUNK = 128
N_CHUNKS = PER_W // CHUNK
SUB = 5
GROUP = SUB * CHUNK
N_GROUPS = PER_W // GROUP
T = N_GROUPS // 2


def _body(table_hbm, idx_hbm, out_hbm, idx_v, buf0, buf1, gsem0, gsem1,
          ssem0, ssem1):
  wid = lax.axis_index("s") * NC + lax.axis_index("c")
  base = wid * PER_W

  pltpu.sync_copy(idx_hbm.at[wid], idx_v)

  def issue_gathers(g, buf, sem):
    for q in range(SUB):
      pltpu.async_copy(table_hbm.at[idx_v.at[g * SUB + q]],
                       buf.at[pl.ds(q * CHUNK, CHUNK), :], sem)

  def wait_gathers(buf, sem):
    for q in range(SUB):
      pltpu.make_async_copy(table_hbm.at[idx_v.at[q]],
                            buf.at[pl.ds(q * CHUNK, CHUNK), :], sem).wait()

  def issue_store(g, buf, sem):
    pltpu.async_copy(buf, out_hbm.at[pl.ds(base + g * GROUP, GROUP), :], sem)

  def wait_store(buf, sem):
    pltpu.make_async_copy(buf, out_hbm.at[pl.ds(base, GROUP), :], sem).wait()

  issue_gathers(0, buf0, gsem0)

  @pl.loop(0, T)
  def _(t):
    a = 2 * t

    @pl.when(t > 0)
    def _():
      wait_store(buf1, ssem1)
    issue_gathers(a + 1, buf1, gsem1)

    wait_gathers(buf0, gsem0)
    issue_store(a, buf0, ssem0)

    @pl.when(t < T - 1)
    def _():
      wait_store(buf0, ssem0)
      issue_gathers(a + 2, buf0, gsem0)

    wait_gathers(buf1, gsem1)
    issue_store(a + 1, buf1, ssem1)

  wait_store(buf0, ssem0)
  wait_store(buf1, ssem1)


@jax.jit
def _lookup(x_flat3, emb_weight):
  mesh = plsc.VectorSubcoreMesh(
      core_axis_name="c", subcore_axis_name="s", num_cores=NC,
      num_subcores=NS)
  scratch = [
      pltpu.VMEM((N_CHUNKS, CHUNK), jnp.int32),
      pltpu.VMEM((GROUP, EMB_DIM), jnp.float32),
      pltpu.VMEM((GROUP, EMB_DIM), jnp.float32),
      pltpu.SemaphoreType.DMA,
      pltpu.SemaphoreType.DMA,
      pltpu.SemaphoreType.DMA,
      pltpu.SemaphoreType.DMA,
  ]
  return pl.kernel(
      _body,
      out_type=jax.ShapeDtypeStruct((B, EMB_DIM), jnp.float32),
      mesh=mesh,
      scratch_types=scratch,
      compiler_params=pltpu.CompilerParams(use_tc_tiling_on_sc=False),
  )(emb_weight, x_flat3)


def kernel(x, emb_weight):
  idx3 = x.astype(jnp.int32).reshape(NW, N_CHUNKS, CHUNK)
  out = _lookup(idx3, emb_weight)
  return out.reshape(BATCH, HIST_LEN, EMB_DIM)
